# Initial kernel scaffold; baseline (speedup 1.0000x reference)
import jax
import jax.numpy as jnp
from jax.experimental import pallas as pl


def kernel(emb1, emb2, lemmas, lemma_embs):
    raise NotImplementedError("write your pallas kernel here")


# speedup vs baseline: 1.6316x; 1.6316x over previous
import functools

import jax
import jax.numpy as jnp
from jax import lax
from jax.experimental import pallas as pl
from jax.experimental.pallas import tpu as pltpu
from jax.experimental.pallas import tpu_sc as plsc

B = 16384
D_IN = 64
D_EMB = 2 * D_IN
NC = 2
NS = 16
NW = NC * NS
SUB = 128
NJ = B // (NW * SUB)
L = 16


def _sc_body(table_hbm, lem_hbm, e1_hbm, e2_hbm, out_hbm,
             idx_v, rows_v, e1_v, e2_v, p_scr, out_v, sem0, sem1):
    wid = lax.axis_index("s") * NC + lax.axis_index("c")
    sems = (sem0, sem1)

    pltpu.sync_copy(lem_hbm.at[wid], idx_v)

    def start(j, b):
        return (
            pltpu.async_copy(table_hbm.at[idx_v.at[j]], rows_v.at[b], sems[b]),
            pltpu.async_copy(e1_hbm.at[wid, j], e1_v.at[b], sems[b]),
            pltpu.async_copy(e2_hbm.at[wid, j], e2_v.at[b], sems[b]),
        )

    lane_iota = lax.broadcasted_iota(jnp.int32, (L,), 0)

    def compute(j, b):
        def group(g, carry):
            base = g * L
            for jj in range(L):
                r = base + jj
                acc = rows_v[b, r, pl.ds(0, L)] * e1_v[b, r, pl.ds(0, L)]
                for k in range(1, 4):
                    acc += rows_v[b, r, pl.ds(k * L, L)] * e1_v[b, r, pl.ds(k * L, L)]
                for k in range(4):
                    acc += (rows_v[b, r, pl.ds(D_IN + k * L, L)]
                            * e2_v[b, r, pl.ds(k * L, L)])
                p_scr[jj, :] = acc
            tot = plsc.load_gather(p_scr, [lane_iota, jnp.zeros((L,), jnp.int32)])
            for d in range(1, L):
                tot += plsc.load_gather(
                    p_scr, [lane_iota, jnp.full((L,), d, jnp.int32)])
            out_v[j, pl.ds(base, L)] = 1.0 / (1.0 + jnp.exp(-tot))
            return carry

        lax.fori_loop(0, SUB // L, group, 0)

    handles = start(0, 0)
    for j in range(NJ):
        b = j % 2
        if j + 1 < NJ:
            next_handles = start(j + 1, (j + 1) % 2)
        for h in handles:
            h.wait()
        compute(j, b)
        if j + 1 < NJ:
            handles = next_handles

    pltpu.sync_copy(out_v, out_hbm.at[wid])


@jax.jit
def _run(lemma_embs, lem_r, e1_r, e2_r):
    mesh = plsc.VectorSubcoreMesh(core_axis_name="c", subcore_axis_name="s")
    f = functools.partial(
        pl.kernel,
        mesh=mesh,
        compiler_params=pltpu.CompilerParams(needs_layout_passes=False),
        out_type=jax.ShapeDtypeStruct((NW, NJ, SUB), jnp.float32),
        scratch_types=[
            pltpu.VMEM((NJ, SUB), jnp.int32),
            pltpu.VMEM((2, SUB, D_EMB), jnp.float32),
            pltpu.VMEM((2, SUB, D_IN), jnp.float32),
            pltpu.VMEM((2, SUB, D_IN), jnp.float32),
            pltpu.VMEM((L, L), jnp.float32),
            pltpu.VMEM((NJ, SUB), jnp.float32),
            pltpu.SemaphoreType.DMA,
            pltpu.SemaphoreType.DMA,
        ],
    )(_sc_body)
    return f(lemma_embs, lem_r, e1_r, e2_r)


def kernel(emb1, emb2, lemmas, lemma_embs):
    lem_r = lemmas.reshape(NW, NJ, SUB)
    e1_r = emb1.reshape(NW, NJ, SUB, D_IN)
    e2_r = emb2.reshape(NW, NJ, SUB, D_IN)
    out = _run(lemma_embs, lem_r, e1_r, e2_r)
    return out.reshape(B)
